# baseline (speedup 1.0000x reference)
import functools

import jax
import jax.numpy as jnp
from jax import lax
from jax.experimental import pallas as pl
from jax.experimental.pallas import tpu as pltpu

_THR = 0.5
_N = 20000
_B = 128
_C = 2048
_NPAD = 20480
_P = _NPAD // _B
_T = _NPAD // _C


def _nms_body(x1_ref, y1_ref, x2_ref, y2_ref, ar_ref, keep_ref,
              supp_ref):
    p = pl.program_id(0)

    @pl.when(p == 0)
    def _init():
        supp_ref[...] = jnp.zeros((_NPAD,), jnp.float32)

    base = p * _B
    px1 = x1_ref[pl.ds(base, _B)]
    py1 = y1_ref[pl.ds(base, _B)]
    px2 = x2_ref[pl.ds(base, _B)]
    py2 = y2_ref[pl.ds(base, _B)]
    par = ar_ref[pl.ds(base, _B)]

    cx1 = px1.reshape(_B, 1)
    cy1 = py1.reshape(_B, 1)
    cx2 = px2.reshape(_B, 1)
    cy2 = py2.reshape(_B, 1)
    car = par.reshape(_B, 1)

    rx1 = px1.reshape(1, _B)
    ry1 = py1.reshape(1, _B)
    rx2 = px2.reshape(1, _B)
    ry2 = py2.reshape(1, _B)
    rar = par.reshape(1, _B)

    iw = jnp.maximum(jnp.minimum(cx2, rx2) - jnp.maximum(cx1, rx1), 0.0)
    ih = jnp.maximum(jnp.minimum(cy2, ry2) - jnp.maximum(cy1, ry1), 0.0)
    inter = iw * ih
    union = (car + rar) - inter
    iou = inter / union
    rows_i = lax.broadcasted_iota(jnp.int32, (_B, _B), 0)
    cols_j = lax.broadcasted_iota(jnp.int32, (_B, _B), 1)
    m = jnp.where((iou > _THR) & (cols_j > rows_i), 1.0, 0.0)

    s0 = supp_ref[pl.ds(base, _B)].reshape(1, _B)
    keep0 = 1.0 - s0

    def jac_cond(c):
        return c[1]

    def jac_body(c):
        keep, _ = c
        cnt = jnp.dot(keep, m, preferred_element_type=jnp.float32)
        keep_new = jnp.where((s0 > 0.0) | (cnt > 0.0), 0.0, 1.0)
        changed = jnp.any(keep_new != keep)
        return keep_new, changed

    keep_row, _ = lax.while_loop(jac_cond, jac_body, (keep0, True))
    s = 1.0 - keep_row
    supp_ref[pl.ds(base, _B)] = s.reshape(_B)
    keep_ref[...] = keep_row.reshape(_B)

    kept_row = keep_row

    t0 = base // _C

    def cross_step(t, _):
      @pl.when(t >= t0)
      def _do():
        cbase = t * _C
        tx1 = x1_ref[pl.ds(cbase, _C)].reshape(1, _C)
        ty1 = y1_ref[pl.ds(cbase, _C)].reshape(1, _C)
        tx2 = x2_ref[pl.ds(cbase, _C)].reshape(1, _C)
        ty2 = y2_ref[pl.ds(cbase, _C)].reshape(1, _C)
        tar = ar_ref[pl.ds(cbase, _C)].reshape(1, _C)
        jw = jnp.maximum(jnp.minimum(cx2, tx2) - jnp.maximum(cx1, tx1), 0.0)
        jh = jnp.maximum(jnp.minimum(cy2, ty2) - jnp.maximum(cy1, ty1), 0.0)
        jinter = jw * jh
        junion = (car + tar) - jinter
        jiou = jinter / junion
        sup = jnp.where(jiou > _THR, 1.0, 0.0)
        cnt = jnp.dot(kept_row, sup, preferred_element_type=jnp.float32)
        pos = cbase + lax.broadcasted_iota(jnp.int32, (1, _C), 1)
        hit = jnp.where((cnt > 0.0) & (pos >= base + _B), 1.0, 0.0)
        old = supp_ref[pl.ds(cbase, _C)]
        supp_ref[pl.ds(cbase, _C)] = jnp.maximum(old, hit.reshape(_C))

      return 0

    lax.fori_loop(0, _T, cross_step, 0, unroll=2)


@jax.jit
def _nms_pallas(sx1, sy1, sx2, sy2, sar):
    full = pl.BlockSpec((_NPAD,), lambda p: (0,))
    return pl.pallas_call(
        _nms_body,
        grid=(_P,),
        in_specs=[full, full, full, full, full],
        out_specs=pl.BlockSpec((_B,), lambda p: (p,)),
        out_shape=jax.ShapeDtypeStruct((_NPAD,), jnp.float32),
        scratch_shapes=[
            pltpu.VMEM((_NPAD,), jnp.float32),
        ],
    )(sx1, sy1, sx2, sy2, sar)


def kernel(boxes, scores):
    order = jnp.argsort(-scores)
    sboxes = boxes[order]
    npad = _NPAD - _N
    pad = jnp.tile(jnp.array([[-3.0, -3.0, -2.0, -2.0]], jnp.float32),
                   (npad, 1))
    sboxes = jnp.concatenate([sboxes, pad], axis=0)
    sx1, sy1, sx2, sy2 = (sboxes[:, 0], sboxes[:, 1],
                          sboxes[:, 2], sboxes[:, 3])
    sar = (sx2 - sx1) * (sy2 - sy1)
    keep_sorted = _nms_pallas(sx1, sy1, sx2, sy2, sar)
    keep_mask = jnp.zeros((_N,), bool).at[order].set(keep_sorted[:_N] > 0.5)
    return keep_mask


# speedup vs baseline: 1.0037x; 1.0037x over previous
import functools

import jax
import jax.numpy as jnp
from jax import lax
from jax.experimental import pallas as pl
from jax.experimental.pallas import tpu as pltpu

_THR = 0.5
_N = 20000
_B = 128
_C = 2048
_NPAD = 20480
_P = _NPAD // _B
_T = _NPAD // _C


def _nms_body(x1_ref, y1_ref, x2_ref, y2_ref, ar_ref, keep_ref,
              supp_ref):
    p = pl.program_id(0)

    @pl.when(p == 0)
    def _init():
        supp_ref[...] = jnp.zeros((_NPAD,), jnp.float32)

    base = p * _B
    px1 = x1_ref[pl.ds(base, _B)]
    py1 = y1_ref[pl.ds(base, _B)]
    px2 = x2_ref[pl.ds(base, _B)]
    py2 = y2_ref[pl.ds(base, _B)]
    par = ar_ref[pl.ds(base, _B)]

    cx1 = px1.reshape(_B, 1)
    cy1 = py1.reshape(_B, 1)
    cx2 = px2.reshape(_B, 1)
    cy2 = py2.reshape(_B, 1)
    car = par.reshape(_B, 1)

    rx1 = px1.reshape(1, _B)
    ry1 = py1.reshape(1, _B)
    rx2 = px2.reshape(1, _B)
    ry2 = py2.reshape(1, _B)
    rar = par.reshape(1, _B)

    iw = jnp.maximum(jnp.minimum(cx2, rx2) - jnp.maximum(cx1, rx1), 0.0)
    ih = jnp.maximum(jnp.minimum(cy2, ry2) - jnp.maximum(cy1, ry1), 0.0)
    inter = iw * ih
    union = (car + rar) - inter
    iou = inter / union
    rows_i = lax.broadcasted_iota(jnp.int32, (_B, _B), 0)
    cols_j = lax.broadcasted_iota(jnp.int32, (_B, _B), 1)
    hit = iou > _THR
    m_up = jnp.where(hit & (cols_j > rows_i), 1.0, 0.0)
    m_lo = jnp.where(hit & (cols_j < rows_i), 1.0, 0.0)

    s0 = supp_ref[pl.ds(base, _B)].reshape(1, _B)
    s0_col = s0.reshape(_B, 1)
    keep0 = 1.0 - s0

    def jac_cond(c):
        return c[1]

    def jac_body(c):
        keep, _ = c
        cnt_col = jnp.sum(m_lo * keep, axis=1, keepdims=True)
        keep_col = jnp.where((s0_col > 0.0) | (cnt_col > 0.0), 0.0, 1.0)
        cnt_row = jnp.sum(m_up * keep_col, axis=0, keepdims=True)
        keep_new = jnp.where((s0 > 0.0) | (cnt_row > 0.0), 0.0, 1.0)
        changed = jnp.any(keep_new != keep)
        return keep_new, changed

    keep_row, _ = lax.while_loop(jac_cond, jac_body, (keep0, True))
    s = 1.0 - keep_row
    supp_ref[pl.ds(base, _B)] = s.reshape(_B)
    keep_ref[...] = keep_row.reshape(_B)

    kept_row = keep_row

    t0 = base // _C

    def cross_step(t, _):
      @pl.when(t >= t0)
      def _do():
        cbase = t * _C
        tx1 = x1_ref[pl.ds(cbase, _C)].reshape(1, _C)
        ty1 = y1_ref[pl.ds(cbase, _C)].reshape(1, _C)
        tx2 = x2_ref[pl.ds(cbase, _C)].reshape(1, _C)
        ty2 = y2_ref[pl.ds(cbase, _C)].reshape(1, _C)
        tar = ar_ref[pl.ds(cbase, _C)].reshape(1, _C)
        jw = jnp.maximum(jnp.minimum(cx2, tx2) - jnp.maximum(cx1, tx1), 0.0)
        jh = jnp.maximum(jnp.minimum(cy2, ty2) - jnp.maximum(cy1, ty1), 0.0)
        jinter = jw * jh
        junion = (car + tar) - jinter
        jiou = jinter / junion
        sup = jnp.where(jiou > _THR, 1.0, 0.0)
        cnt = jnp.dot(kept_row, sup, preferred_element_type=jnp.float32)
        pos = cbase + lax.broadcasted_iota(jnp.int32, (1, _C), 1)
        hit = jnp.where((cnt > 0.0) & (pos >= base + _B), 1.0, 0.0)
        old = supp_ref[pl.ds(cbase, _C)]
        supp_ref[pl.ds(cbase, _C)] = jnp.maximum(old, hit.reshape(_C))

      return 0

    lax.fori_loop(0, _T, cross_step, 0, unroll=2)


@jax.jit
def _nms_pallas(sx1, sy1, sx2, sy2, sar):
    full = pl.BlockSpec((_NPAD,), lambda p: (0,))
    return pl.pallas_call(
        _nms_body,
        grid=(_P,),
        in_specs=[full, full, full, full, full],
        out_specs=pl.BlockSpec((_B,), lambda p: (p,)),
        out_shape=jax.ShapeDtypeStruct((_NPAD,), jnp.float32),
        scratch_shapes=[
            pltpu.VMEM((_NPAD,), jnp.float32),
        ],
    )(sx1, sy1, sx2, sy2, sar)


def kernel(boxes, scores):
    order = jnp.argsort(-scores)
    sboxes = boxes[order]
    npad = _NPAD - _N
    pad = jnp.tile(jnp.array([[-3.0, -3.0, -2.0, -2.0]], jnp.float32),
                   (npad, 1))
    sboxes = jnp.concatenate([sboxes, pad], axis=0)
    sx1, sy1, sx2, sy2 = (sboxes[:, 0], sboxes[:, 1],
                          sboxes[:, 2], sboxes[:, 3])
    sar = (sx2 - sx1) * (sy2 - sy1)
    keep_sorted = _nms_pallas(sx1, sy1, sx2, sy2, sar)
    keep_mask = jnp.zeros((_N,), bool).at[order].set(keep_sorted[:_N] > 0.5)
    return keep_mask


# speedup vs baseline: 1.0227x; 1.0189x over previous
import functools

import jax
import jax.numpy as jnp
from jax import lax
from jax.experimental import pallas as pl
from jax.experimental.pallas import tpu as pltpu

_THR = 0.5
_N = 20000
_B = 128
_C = 4096
_NPAD = 20480
_P = _NPAD // _B
_T = _NPAD // _C


def _nms_body(x1_ref, y1_ref, x2_ref, y2_ref, ar_ref, keep_ref,
              supp_ref):
    p = pl.program_id(0)

    @pl.when(p == 0)
    def _init():
        supp_ref[...] = jnp.zeros((_NPAD,), jnp.float32)

    base = p * _B
    px1 = x1_ref[pl.ds(base, _B)]
    py1 = y1_ref[pl.ds(base, _B)]
    px2 = x2_ref[pl.ds(base, _B)]
    py2 = y2_ref[pl.ds(base, _B)]
    par = ar_ref[pl.ds(base, _B)]

    cx1 = px1.reshape(_B, 1)
    cy1 = py1.reshape(_B, 1)
    cx2 = px2.reshape(_B, 1)
    cy2 = py2.reshape(_B, 1)
    car = par.reshape(_B, 1)

    rx1 = px1.reshape(1, _B)
    ry1 = py1.reshape(1, _B)
    rx2 = px2.reshape(1, _B)
    ry2 = py2.reshape(1, _B)
    rar = par.reshape(1, _B)

    iw = jnp.maximum(jnp.minimum(cx2, rx2) - jnp.maximum(cx1, rx1), 0.0)
    ih = jnp.maximum(jnp.minimum(cy2, ry2) - jnp.maximum(cy1, ry1), 0.0)
    inter = iw * ih
    union = (car + rar) - inter
    iou = inter / union
    rows_i = lax.broadcasted_iota(jnp.int32, (_B, _B), 0)
    cols_j = lax.broadcasted_iota(jnp.int32, (_B, _B), 1)
    hit = iou > _THR
    m_up = jnp.where(hit & (cols_j > rows_i), 1.0, 0.0)
    m_lo = jnp.where(hit & (cols_j < rows_i), 1.0, 0.0)

    s0 = supp_ref[pl.ds(base, _B)].reshape(1, _B)
    s0_col = s0.reshape(_B, 1)
    keep0 = 1.0 - s0

    def jac_cond(c):
        return c[1]

    def jac_body(c):
        keep, _ = c
        cnt_col = jnp.sum(m_lo * keep, axis=1, keepdims=True)
        keep_col = jnp.where((s0_col > 0.0) | (cnt_col > 0.0), 0.0, 1.0)
        cnt_row = jnp.sum(m_up * keep_col, axis=0, keepdims=True)
        keep_new = jnp.where((s0 > 0.0) | (cnt_row > 0.0), 0.0, 1.0)
        changed = jnp.any(keep_new != keep)
        return keep_new, changed

    keep_row, _ = lax.while_loop(jac_cond, jac_body, (keep0, True))
    s = 1.0 - keep_row
    supp_ref[pl.ds(base, _B)] = s.reshape(_B)
    keep_ref[...] = keep_row.reshape(_B)

    kept_row = keep_row

    t0 = base // _C

    def cross_step(t, _):
      @pl.when(t >= t0)
      def _do():
        cbase = t * _C
        tx1 = x1_ref[pl.ds(cbase, _C)].reshape(1, _C)
        ty1 = y1_ref[pl.ds(cbase, _C)].reshape(1, _C)
        tx2 = x2_ref[pl.ds(cbase, _C)].reshape(1, _C)
        ty2 = y2_ref[pl.ds(cbase, _C)].reshape(1, _C)
        tar = ar_ref[pl.ds(cbase, _C)].reshape(1, _C)
        jw = jnp.maximum(jnp.minimum(cx2, tx2) - jnp.maximum(cx1, tx1), 0.0)
        jh = jnp.maximum(jnp.minimum(cy2, ty2) - jnp.maximum(cy1, ty1), 0.0)
        jinter = jw * jh
        junion = (car + tar) - jinter
        jiou = jinter / junion
        sup = jnp.where(jiou > _THR, 1.0, 0.0)
        cnt = jnp.dot(kept_row, sup, preferred_element_type=jnp.float32)
        pos = cbase + lax.broadcasted_iota(jnp.int32, (1, _C), 1)
        hit = jnp.where((cnt > 0.0) & (pos >= base + _B), 1.0, 0.0)
        old = supp_ref[pl.ds(cbase, _C)]
        supp_ref[pl.ds(cbase, _C)] = jnp.maximum(old, hit.reshape(_C))

      return 0

    lax.fori_loop(0, _T, cross_step, 0, unroll=1)


@jax.jit
def _nms_pallas(sx1, sy1, sx2, sy2, sar):
    full = pl.BlockSpec((_NPAD,), lambda p: (0,))
    return pl.pallas_call(
        _nms_body,
        grid=(_P,),
        in_specs=[full, full, full, full, full],
        out_specs=pl.BlockSpec((_B,), lambda p: (p,)),
        out_shape=jax.ShapeDtypeStruct((_NPAD,), jnp.float32),
        scratch_shapes=[
            pltpu.VMEM((_NPAD,), jnp.float32),
        ],
    )(sx1, sy1, sx2, sy2, sar)


def kernel(boxes, scores):
    order = jnp.argsort(-scores)
    sboxes = boxes[order]
    npad = _NPAD - _N
    pad = jnp.tile(jnp.array([[-3.0, -3.0, -2.0, -2.0]], jnp.float32),
                   (npad, 1))
    sboxes = jnp.concatenate([sboxes, pad], axis=0)
    sx1, sy1, sx2, sy2 = (sboxes[:, 0], sboxes[:, 1],
                          sboxes[:, 2], sboxes[:, 3])
    sar = (sx2 - sx1) * (sy2 - sy1)
    keep_sorted = _nms_pallas(sx1, sy1, sx2, sy2, sar)
    keep_mask = jnp.zeros((_N,), bool).at[order].set(keep_sorted[:_N] > 0.5)
    return keep_mask


# speedup vs baseline: 1.0958x; 1.0715x over previous
import functools

import jax
import jax.numpy as jnp
from jax import lax
from jax.experimental import pallas as pl
from jax.experimental.pallas import tpu as pltpu

_THR = 0.5
_N = 20000
_B = 256
_C = 4096
_NPAD = 20480
_P = _NPAD // _B
_T = _NPAD // _C


def _nms_body(x1_ref, y1_ref, x2_ref, y2_ref, ar_ref, keep_ref,
              supp_ref):
    p = pl.program_id(0)

    @pl.when(p == 0)
    def _init():
        supp_ref[...] = jnp.zeros((_NPAD,), jnp.float32)

    base = p * _B
    px1 = x1_ref[pl.ds(base, _B)]
    py1 = y1_ref[pl.ds(base, _B)]
    px2 = x2_ref[pl.ds(base, _B)]
    py2 = y2_ref[pl.ds(base, _B)]
    par = ar_ref[pl.ds(base, _B)]

    cx1 = px1.reshape(_B, 1)
    cy1 = py1.reshape(_B, 1)
    cx2 = px2.reshape(_B, 1)
    cy2 = py2.reshape(_B, 1)
    car = par.reshape(_B, 1)

    rx1 = px1.reshape(1, _B)
    ry1 = py1.reshape(1, _B)
    rx2 = px2.reshape(1, _B)
    ry2 = py2.reshape(1, _B)
    rar = par.reshape(1, _B)

    iw = jnp.maximum(jnp.minimum(cx2, rx2) - jnp.maximum(cx1, rx1), 0.0)
    ih = jnp.maximum(jnp.minimum(cy2, ry2) - jnp.maximum(cy1, ry1), 0.0)
    inter = iw * ih
    union = (car + rar) - inter
    iou = inter / union
    rows_i = lax.broadcasted_iota(jnp.int32, (_B, _B), 0)
    cols_j = lax.broadcasted_iota(jnp.int32, (_B, _B), 1)
    hit = iou > _THR
    m_up = jnp.where(hit & (cols_j > rows_i), 1.0, 0.0)
    m_lo = jnp.where(hit & (cols_j < rows_i), 1.0, 0.0)

    s0 = supp_ref[pl.ds(base, _B)].reshape(1, _B)
    s0_col = s0.reshape(_B, 1)
    keep0 = 1.0 - s0

    def jac_cond(c):
        return c[1]

    def jac_body(c):
        keep, _ = c
        cnt_col = jnp.sum(m_lo * keep, axis=1, keepdims=True)
        keep_col = jnp.where((s0_col > 0.0) | (cnt_col > 0.0), 0.0, 1.0)
        cnt_row = jnp.sum(m_up * keep_col, axis=0, keepdims=True)
        keep_new = jnp.where((s0 > 0.0) | (cnt_row > 0.0), 0.0, 1.0)
        changed = jnp.any(keep_new != keep)
        return keep_new, changed

    keep_row, _ = lax.while_loop(jac_cond, jac_body, (keep0, True))
    s = 1.0 - keep_row
    supp_ref[pl.ds(base, _B)] = s.reshape(_B)
    keep_ref[...] = keep_row.reshape(_B)

    kept_row = keep_row

    t0 = base // _C

    def cross_step(t, _):
      @pl.when(t >= t0)
      def _do():
        cbase = t * _C
        tx1 = x1_ref[pl.ds(cbase, _C)].reshape(1, _C)
        ty1 = y1_ref[pl.ds(cbase, _C)].reshape(1, _C)
        tx2 = x2_ref[pl.ds(cbase, _C)].reshape(1, _C)
        ty2 = y2_ref[pl.ds(cbase, _C)].reshape(1, _C)
        tar = ar_ref[pl.ds(cbase, _C)].reshape(1, _C)
        jw = jnp.maximum(jnp.minimum(cx2, tx2) - jnp.maximum(cx1, tx1), 0.0)
        jh = jnp.maximum(jnp.minimum(cy2, ty2) - jnp.maximum(cy1, ty1), 0.0)
        jinter = jw * jh
        junion = (car + tar) - jinter
        jiou = jinter / junion
        sup = jnp.where(jiou > _THR, 1.0, 0.0)
        cnt = jnp.dot(kept_row, sup, preferred_element_type=jnp.float32)
        pos = cbase + lax.broadcasted_iota(jnp.int32, (1, _C), 1)
        hit = jnp.where((cnt > 0.0) & (pos >= base + _B), 1.0, 0.0)
        old = supp_ref[pl.ds(cbase, _C)]
        supp_ref[pl.ds(cbase, _C)] = jnp.maximum(old, hit.reshape(_C))

      return 0

    lax.fori_loop(0, _T, cross_step, 0, unroll=1)


@jax.jit
def _nms_pallas(sx1, sy1, sx2, sy2, sar):
    full = pl.BlockSpec((_NPAD,), lambda p: (0,))
    return pl.pallas_call(
        _nms_body,
        grid=(_P,),
        in_specs=[full, full, full, full, full],
        out_specs=pl.BlockSpec((_B,), lambda p: (p,)),
        out_shape=jax.ShapeDtypeStruct((_NPAD,), jnp.float32),
        scratch_shapes=[
            pltpu.VMEM((_NPAD,), jnp.float32),
        ],
    )(sx1, sy1, sx2, sy2, sar)


def kernel(boxes, scores):
    order = jnp.argsort(-scores)
    sboxes = boxes[order]
    npad = _NPAD - _N
    pad = jnp.tile(jnp.array([[-3.0, -3.0, -2.0, -2.0]], jnp.float32),
                   (npad, 1))
    sboxes = jnp.concatenate([sboxes, pad], axis=0)
    sx1, sy1, sx2, sy2 = (sboxes[:, 0], sboxes[:, 1],
                          sboxes[:, 2], sboxes[:, 3])
    sar = (sx2 - sx1) * (sy2 - sy1)
    keep_sorted = _nms_pallas(sx1, sy1, sx2, sy2, sar)
    keep_mask = jnp.zeros((_N,), bool).at[order].set(keep_sorted[:_N] > 0.5)
    return keep_mask
